# Initial kernel scaffold; baseline (speedup 1.0000x reference)
import jax
import jax.numpy as jnp
from jax.experimental import pallas as pl


def kernel(inputs, W):
    raise NotImplementedError("write your pallas kernel here")


# speedup vs baseline: 7.3308x; 7.3308x over previous
import jax
import jax.numpy as jnp
from jax.experimental import pallas as pl
from jax.experimental.pallas import tpu as pltpu

_NUM_E = 8192
_DIM = 64
_N_TOK = 8192
_BLK = 256
_NBLK = _N_TOK // _BLK
_COMMIT = 0.25


def _vq_body(x_ref, w_ref, idx_ref,
             enc_ref, q_ref, counts_ref, loss_ref, perp_ref):
    i = pl.program_id(0)

    @pl.when(i == 0)
    def _init():
        counts_ref[...] = jnp.zeros_like(counts_ref)
        loss_ref[...] = jnp.zeros_like(loss_ref)
        perp_ref[...] = jnp.zeros_like(perp_ref)

    x = x_ref[...]
    w = w_ref[...]
    idx = idx_ref[...]

    iota = jax.lax.broadcasted_iota(jnp.int32, (_BLK, _NUM_E), 1)
    onehot = (iota == idx).astype(jnp.float32)

    enc_ref[...] = onehot
    q = jax.lax.dot_general(onehot, w, (((1,), (0,)), ((), ())))
    q_ref[...] = x + (q - x)

    counts_ref[...] += jnp.sum(onehot, axis=0, keepdims=True)
    loss_ref[...] += jnp.sum((q - x) ** 2).reshape(1, 1)

    @pl.when(i == _NBLK - 1)
    def _fin():
        loss_ref[...] = _COMMIT * loss_ref[...] / (_N_TOK * _DIM)
        p = counts_ref[...] / _N_TOK
        perp_ref[...] = jnp.exp(-jnp.sum(p * jnp.log(p + 1e-10))).reshape(1, 1)


@jax.jit
def kernel(inputs, W):
    flat_input = inputs
    distances = (
        jnp.sum(flat_input ** 2, axis=1, keepdims=True)
        + jnp.sum(W ** 2, axis=1)
        - 2.0 * jnp.matmul(flat_input, W.T)
    )
    encoding_indices = jnp.argmin(distances, axis=1)
    idx = encoding_indices.astype(jnp.int32)[:, None]

    out_shapes = (
        jax.ShapeDtypeStruct((_N_TOK, _NUM_E), jnp.float32),
        jax.ShapeDtypeStruct((_N_TOK, _DIM), jnp.float32),
        jax.ShapeDtypeStruct((1, _NUM_E), jnp.float32),
        jax.ShapeDtypeStruct((1, 1), jnp.float32),
        jax.ShapeDtypeStruct((1, 1), jnp.float32),
    )
    enc, q, _counts, loss, perp = pl.pallas_call(
        _vq_body,
        grid=(_NBLK,),
        in_specs=[
            pl.BlockSpec((_BLK, _DIM), lambda i: (i, 0)),
            pl.BlockSpec((_NUM_E, _DIM), lambda i: (0, 0)),
            pl.BlockSpec((_BLK, 1), lambda i: (i, 0)),
        ],
        out_specs=(
            pl.BlockSpec((_BLK, _NUM_E), lambda i: (i, 0)),
            pl.BlockSpec((_BLK, _DIM), lambda i: (i, 0)),
            pl.BlockSpec((1, _NUM_E), lambda i: (0, 0)),
            pl.BlockSpec((1, 1), lambda i: (0, 0)),
            pl.BlockSpec((1, 1), lambda i: (0, 0)),
        ),
        out_shape=out_shapes,
        compiler_params=pltpu.CompilerParams(
            dimension_semantics=("arbitrary",),
        ),
    )(inputs, W, idx)
    return (loss[0, 0], q, perp[0, 0], enc)
